# baseline (speedup 1.0000x reference)
import functools

import numpy as np

import jax
import jax.numpy as jnp
from jax import lax
from jax.experimental import pallas as pl
from jax.experimental.pallas import tpu as pltpu
from jax.experimental.pallas import tpu_sc as plsc

NC = 2
NS = 16
NW = NC * NS
P = 128
HB = 3
NB = 512
LANES = 16


def _y_matmul(x, w2, brow, K, D):
    B, C, N = x.shape
    KO = w2.shape[1]
    npad16 = -(-N // 16) * 16
    nblk = pl.cdiv(npad16, NB)

    def body(x_ref, w_ref, b_ref, y_ref):
        xb = x_ref[...].reshape(B * C, NB)
        for j in range((K + 1) // 2):
            c0 = 2 * j * D
            cw = min(2 * D, KO - c0)
            y = lax.dot_general(
                xb, w_ref[:, c0:c0 + cw], (((0,), (0,)), ((), ())),
                preferred_element_type=jnp.float32) + b_ref[:, c0:c0 + cw]
            y_ref[2 * j] = y[:, :D]
            if cw > D:
                y_ref[2 * j + 1] = y[:, D:]

    return pl.pallas_call(
        body,
        grid=(nblk,),
        in_specs=[
            pl.BlockSpec((B, C, NB), lambda i: (0, 0, i)),
            pl.BlockSpec((B * C, KO), lambda i: (0, 0)),
            pl.BlockSpec((1, KO), lambda i: (0, 0)),
        ],
        out_specs=pl.BlockSpec((K, NB, D), lambda i: (0, i, 0)),
        out_shape=jax.ShapeDtypeStruct((K, npad16, D), jnp.float32),
    )(x, w2, brow)


def _transpose_out(z, B, OUT, N):
    D = z.shape[1]
    NT = 2048

    def body(z_ref, o_ref):
        o_ref[...] = jnp.transpose(z_ref[...]).reshape(B, OUT, NT)

    return pl.pallas_call(
        body,
        grid=(pl.cdiv(N, NT),),
        in_specs=[pl.BlockSpec((NT, D), lambda i: (i, 0))],
        out_specs=pl.BlockSpec((B, OUT, NT), lambda i: (0, 0, i)),
        out_shape=jax.ShapeDtypeStruct((B, OUT, N), jnp.float32),
    )(z)


def _sc_gather_sum(y2, idx3, idxt, K, D, G, P2, npad, init=None):
    ngrp = -(-K // HB)
    lo = K // ngrp
    groups = [lo + (1 if i < K - lo * ngrp else 0) for i in range(ngrp)]
    koffs = [sum(groups[:i]) for i in range(ngrp)]

    mesh = plsc.VectorSubcoreMesh(
        core_axis_name="c", subcore_axis_name="s",
        num_cores=NC, num_subcores=NS)
    has_init = init is not None
    inputs = (y2, idx3, idxt) + ((init,) if has_init else ())

    @functools.partial(
        pl.kernel,
        out_type=jax.ShapeDtypeStruct((npad, D), jnp.float32),
        mesh=mesh,
        scratch_types=[
            pltpu.VMEM((K, P), jnp.int32),
            pltpu.VMEM((2, HB, P, D), jnp.float32),
            pltpu.VMEM((P, D), jnp.float32),
            pltpu.SemaphoreType.DMA,
            pltpu.SemaphoreType.DMA,
        ],
    )
    def k(y2_hbm, idx_hbm, idxt_hbm, *rest):
        if has_init:
            init_hbm, out_hbm, idx_v, bufs, acc_v, sem0, sem1 = rest
        else:
            out_hbm, idx_v, bufs, acc_v, sem0, sem1 = rest
            init_hbm = None
        cid = lax.axis_index("c")
        sid = lax.axis_index("s")
        wid = sid * NC + cid
        sems = [sem0, sem1]

        def do_chunk(pcur):
            def fire(gi):
                bank = gi % 2
                return [
                    pltpu.async_copy(
                        y2_hbm.at[idx_v.at[koffs[gi] + j, pl.ds(0, pcur)]],
                        bufs.at[bank, j, pl.ds(0, pcur)], sems[bank])
                    for j in range(groups[gi])
                ]

            def run(base):
                handles = fire(0)
                if has_init:
                    pltpu.sync_copy(init_hbm.at[pl.ds(base, pcur)],
                                    acc_v.at[pl.ds(0, pcur)])
                for gi, gsz in enumerate(groups):
                    bank = gi % 2
                    for h in handles:
                        h.wait()
                    if gi + 1 < ngrp:
                        handles = fire(gi + 1)

                    def accum(p, c2):
                        for cc in range(D // LANES):
                            sl = pl.ds(cc * LANES, LANES)
                            s = acc_v[p, sl] if (gi > 0 or has_init) else None
                            for j in range(gsz):
                                g = bufs[bank, j, p, sl]
                                s = g if s is None else s + g
                            acc_v[p, sl] = s
                        return c2

                    lax.fori_loop(0, pcur, accum, 0)
                pltpu.sync_copy(acc_v.at[pl.ds(0, pcur)],
                                out_hbm.at[pl.ds(base, pcur)])
            return run

        main_chunk = do_chunk(P)

        def chunk(g, carry):
            t = g * NW + wid
            pltpu.sync_copy(idx_hbm.at[t], idx_v)
            main_chunk(t * P)
            return carry

        lax.fori_loop(0, G, chunk, 0)

        if P2 > 0:
            tail_chunk = do_chunk(P2)

            @pl.when(wid == 0)
            def _():
                pltpu.sync_copy(idxt_hbm, idx_v)
                tail_chunk(G * NW * P)

    return k(*inputs)


def kernel(x, neigh_orders, W, b):
    B, C, N = x.shape
    K = neigh_orders.shape[1]
    OUT = W.shape[0]
    D = B * OUT

    wr = W.reshape(OUT, K, C).transpose(2, 1, 0)
    eyeb = jnp.eye(B, dtype=W.dtype)
    w2 = (wr[None, :, :, None, :] * eyeb[:, None, None, :, None]
          ).reshape(B * C, K * D)
    brow = jnp.tile(jnp.tile(b, B) / K, K)[None, :]

    KA = min(7, K - 1) if K > 1 else K
    KB = K - KA
    npad16 = -(-N // 16) * 16

    G = N // (NW * P)
    nmain = G * NW * P
    rem = N - nmain
    P2 = -(-rem // 8) * 8 if rem else 0
    npad = nmain + P2

    def make_idx(neigh_cols, nk):
        idx = neigh_cols.astype(jnp.int32) + \
            npad16 * jnp.arange(nk, dtype=jnp.int32)[None, :]
        idx3 = idx[:nmain].reshape(nmain // P, P, nk).transpose(0, 2, 1)
        idxt = jnp.pad(idx[nmain:], ((0, P - rem), (0, 0))).T
        return idx3, idxt

    yA = _y_matmul(x, w2[:, :KA * D], brow[:, :KA * D], KA, D)
    idx3A, idxtA = make_idx(neigh_orders[:, :KA], KA)
    part = _sc_gather_sum(yA.reshape(KA * npad16, D), idx3A, idxtA,
                          KA, D, G, P2, npad)

    yB = _y_matmul(x, w2[:, KA * D:], brow[:, KA * D:], KB, D)
    idx3B, idxtB = make_idx(neigh_orders[:, KA:], KB)
    out = _sc_gather_sum(yB.reshape(KB * npad16, D), idx3B, idxtB,
                         KB, D, G, P2, npad, init=part)

    return _transpose_out(out, B, OUT, N)


# speedup vs baseline: 1.0905x; 1.0905x over previous
import functools

import numpy as np

import jax
import jax.numpy as jnp
from jax import lax
from jax.experimental import pallas as pl
from jax.experimental.pallas import tpu as pltpu
from jax.experimental.pallas import tpu_sc as plsc

NC = 2
NS = 16
NW = NC * NS
P = 128
HB = 3
NB = 512
LANES = 16


def _y_matmul(x, w2, brow, K, D):
    B, C, N = x.shape
    KO = w2.shape[1]
    npad16 = -(-N // 16) * 16
    nblk = pl.cdiv(npad16, NB)

    def body(x_ref, w_ref, b_ref, y_ref):
        xb = x_ref[...].reshape(B * C, NB)
        for j in range((K + 1) // 2):
            c0 = 2 * j * D
            cw = min(2 * D, KO - c0)
            y = lax.dot_general(
                xb, w_ref[:, c0:c0 + cw], (((0,), (0,)), ((), ())),
                preferred_element_type=jnp.float32) + b_ref[:, c0:c0 + cw]
            y_ref[2 * j] = y[:, :D]
            if cw > D:
                y_ref[2 * j + 1] = y[:, D:]

    return pl.pallas_call(
        body,
        grid=(nblk,),
        in_specs=[
            pl.BlockSpec((B, C, NB), lambda i: (0, 0, i)),
            pl.BlockSpec((B * C, KO), lambda i: (0, 0)),
            pl.BlockSpec((1, KO), lambda i: (0, 0)),
        ],
        out_specs=pl.BlockSpec((K, NB, D), lambda i: (0, i, 0)),
        out_shape=jax.ShapeDtypeStruct((K, npad16, D), jnp.float32),
    )(x, w2, brow)


def _transpose_out(z, B, OUT, N):
    D = z.shape[1]
    NT = 2048

    def body(z_ref, o_ref):
        o_ref[...] = jnp.transpose(z_ref[...]).reshape(B, OUT, NT)

    return pl.pallas_call(
        body,
        grid=(pl.cdiv(N, NT),),
        in_specs=[pl.BlockSpec((NT, D), lambda i: (i, 0))],
        out_specs=pl.BlockSpec((B, OUT, NT), lambda i: (0, 0, i)),
        out_shape=jax.ShapeDtypeStruct((B, OUT, N), jnp.float32),
    )(z)


def _sc_gather_sum(y2, idx3, idxt, K, D, G, P2, npad):
    ngrp = -(-K // HB)
    lo = K // ngrp
    groups = [lo + (1 if i < K - lo * ngrp else 0) for i in range(ngrp)]
    koffs = [sum(groups[:i]) for i in range(ngrp)]

    mesh = plsc.VectorSubcoreMesh(
        core_axis_name="c", subcore_axis_name="s",
        num_cores=NC, num_subcores=NS)

    @functools.partial(
        pl.kernel,
        out_type=jax.ShapeDtypeStruct((npad, D), jnp.float32),
        mesh=mesh,
        scratch_types=[
            pltpu.VMEM((K, P), jnp.int32),
            pltpu.VMEM((2, HB, P, D), jnp.float32),
            pltpu.VMEM((P, D), jnp.float32),
            pltpu.SemaphoreType.DMA,
            pltpu.SemaphoreType.DMA,
            pltpu.SemaphoreType.DMA,
        ],
    )
    def k(y2_hbm, idx_hbm, idxt_hbm, out_hbm, idx_v, bufs, acc_v,
          sem0, sem1, sem_out):
        cid = lax.axis_index("c")
        sid = lax.axis_index("s")
        wid = sid * NC + cid
        sems = [sem0, sem1]

        def drain_out():
            pltpu.make_async_copy(y2_hbm.at[pl.ds(0, P)], acc_v,
                                  sem_out).wait()

        def do_chunk(pcur, async_out):
            def fire(gi):
                bank = gi % 2
                return [
                    pltpu.async_copy(
                        y2_hbm.at[idx_v.at[koffs[gi] + j, pl.ds(0, pcur)]],
                        bufs.at[bank, j, pl.ds(0, pcur)], sems[bank])
                    for j in range(groups[gi])
                ]

            def run(base, pending_out):
                handles = fire(0)
                if pending_out:
                    drain_out()
                for gi, gsz in enumerate(groups):
                    bank = gi % 2
                    for h in handles:
                        h.wait()
                    if gi + 1 < ngrp:
                        handles = fire(gi + 1)

                    def accum(p, c2):
                        for cc in range(D // LANES):
                            sl = pl.ds(cc * LANES, LANES)
                            s = None
                            for j in range(gsz):
                                g = bufs[bank, j, p, sl]
                                s = g if s is None else s + g
                            if gi == 0:
                                acc_v[p, sl] = s
                            else:
                                plsc.addupdate(acc_v.at[p, sl], s)
                        return c2

                    lax.fori_loop(0, pcur, accum, 0)
                if async_out:
                    pltpu.async_copy(acc_v.at[pl.ds(0, pcur)],
                                     out_hbm.at[pl.ds(base, pcur)], sem_out)
                else:
                    pltpu.sync_copy(acc_v.at[pl.ds(0, pcur)],
                                    out_hbm.at[pl.ds(base, pcur)])
            return run

        main_chunk = do_chunk(P, True)

        def chunk(g, carry):
            t = g * NW + wid
            pltpu.sync_copy(idx_hbm.at[t], idx_v)

            @pl.when(g == 0)
            def _():
                main_chunk(t * P, False)

            @pl.when(g > 0)
            def _():
                main_chunk(t * P, True)

            return carry

        lax.fori_loop(0, G, chunk, 0)
        if G > 0:
            drain_out()

        if P2 > 0:
            tail_chunk = do_chunk(P2, False)

            @pl.when(wid == 0)
            def _():
                pltpu.sync_copy(idxt_hbm, idx_v)
                tail_chunk(G * NW * P, False)

    return k(y2, idx3, idxt)


def kernel(x, neigh_orders, W, b):
    B, C, N = x.shape
    K = neigh_orders.shape[1]
    OUT = W.shape[0]
    D = B * OUT

    wr = W.reshape(OUT, K, C).transpose(2, 1, 0)
    eyeb = jnp.eye(B, dtype=W.dtype)
    w2 = (wr[None, :, :, None, :] * eyeb[:, None, None, :, None]
          ).reshape(B * C, K * D)
    brow = jnp.tile(jnp.tile(b, B) / K, K)[None, :]

    npad16 = -(-N // 16) * 16
    y = _y_matmul(x, w2, brow, K, D)
    y2 = y.reshape(K * npad16, D)

    idx = neigh_orders.astype(jnp.int32) + npad16 * jnp.arange(K, dtype=jnp.int32)[None, :]

    G = N // (NW * P)
    nmain = G * NW * P
    rem = N - nmain
    P2 = -(-rem // 8) * 8 if rem else 0
    npad = nmain + P2

    idx3 = idx[:nmain].reshape(nmain // P, P, K).transpose(0, 2, 1)
    idxt = jnp.pad(idx[nmain:], ((0, P - rem), (0, 0))).T

    out = _sc_gather_sum(y2, idx3, idxt, K, D, G, P2, npad)

    return _transpose_out(out, B, OUT, N)
